# Initial kernel scaffold; baseline (speedup 1.0000x reference)
import jax
import jax.numpy as jnp
from jax.experimental import pallas as pl


def kernel(U, V, u, pos, neg):
    raise NotImplementedError("write your pallas kernel here")


# speedup vs baseline: 2.8892x; 2.8892x over previous
import functools

import jax
import jax.numpy as jnp
from jax import lax
from jax.experimental import pallas as pl
from jax.experimental.pallas import tpu as pltpu
from jax.experimental.pallas import tpu_sc as plsc

B = 16384
D = 64
K = 10
L = 16
NC = 2
NS = 16
NW = NC * NS
PER_W = B // NW
C = 128
NCHUNK = PER_W // C


def _sc_body(u_tab, v_tab, u_idx, pos_idx, neg_idx, pos_out, neg_out,
             u_idx_v, pos_idx_v, neg_idx_v, u_rows, pos_rows, neg_rows,
             pos_acc, neg_acc, sem):
  wid = lax.axis_index("s") * NC + lax.axis_index("c")
  base = wid * PER_W

  for c in range(NCHUNK):
    off = base + c * C
    ci = wid * NCHUNK + c

    pltpu.sync_copy(u_idx.at[pl.ds(off, C)], u_idx_v)
    pltpu.sync_copy(pos_idx.at[pl.ds(off, C)], pos_idx_v)
    pltpu.sync_copy(neg_idx.at[ci], neg_idx_v)

    cps = [
        pltpu.async_copy(u_tab.at[u_idx_v], u_rows, sem),
        pltpu.async_copy(v_tab.at[pos_idx_v], pos_rows, sem),
    ]
    for j in range(K):
      cps.append(
          pltpu.async_copy(v_tab.at[neg_idx_v.at[j]],
                           neg_rows.at[pl.ds(j * C, C)], sem))
    for cp in cps:
      cp.wait()

    def elem(i, _):
      us = [u_rows[i, pl.ds(d * L, L)] for d in range(D // L)]
      pa = us[0] * pos_rows[i, pl.ds(0, L)]
      for d in range(1, D // L):
        pa = pa + us[d] * pos_rows[i, pl.ds(d * L, L)]
      pos_acc[i, :] = pa
      na = jnp.zeros((L,), jnp.float32)
      for k in range(K):
        r = i * K + k
        for d in range(D // L):
          na = na + us[d] * neg_rows[r, pl.ds(d * L, L)]
      neg_acc[i, :] = na
      return 0

    lax.fori_loop(0, C, elem, 0)

    pltpu.sync_copy(pos_acc, pos_out.at[pl.ds(off, C)])
    pltpu.sync_copy(neg_acc, neg_out.at[pl.ds(off, C)])


_sc_call = pl.kernel(
    _sc_body,
    out_type=(
        jax.ShapeDtypeStruct((B, L), jnp.float32),
        jax.ShapeDtypeStruct((B, L), jnp.float32),
    ),
    mesh=plsc.VectorSubcoreMesh(core_axis_name="c", subcore_axis_name="s"),
    scratch_types=[
        pltpu.VMEM((C,), jnp.int32),
        pltpu.VMEM((C,), jnp.int32),
        pltpu.VMEM((K, C), jnp.int32),
        pltpu.VMEM((C, D), jnp.float32),
        pltpu.VMEM((C, D), jnp.float32),
        pltpu.VMEM((K * C, D), jnp.float32),
        pltpu.VMEM((C, L), jnp.float32),
        pltpu.VMEM((C, L), jnp.float32),
        pltpu.SemaphoreType.DMA,
    ],
    compiler_params=pltpu.CompilerParams(use_tc_tiling_on_sc=False),
)


def _tc_body(pos_ref, neg_ref, out_ref):
  p = jnp.sum(pos_ref[...], axis=1)
  n = jnp.sum(neg_ref[...], axis=1)

  def softplus(x):
    return jnp.maximum(x, 0.0) + jnp.log1p(jnp.exp(-jnp.abs(x)))

  loss = softplus(-p) + softplus(n)
  out_ref[0, 0] = jnp.sum(loss) * (1.0 / B)


_tc_call = pl.pallas_call(
    _tc_body,
    out_shape=jax.ShapeDtypeStruct((1, 1), jnp.float32),
    out_specs=pl.BlockSpec(memory_space=pltpu.SMEM),
)


@jax.jit
def kernel(U, V, u, pos, neg):
  neg2d = neg.reshape(NW * NCHUNK, K, C)
  pos_acc, neg_acc = _sc_call(U, V, u, pos, neg2d)
  loss = _tc_call(pos_acc, neg_acc)
  return loss[0, 0]
